# Initial kernel scaffold; baseline (speedup 1.0000x reference)
import jax
import jax.numpy as jnp
from jax.experimental import pallas as pl


def kernel(inputs, token_table, position_table):
    raise NotImplementedError("write your pallas kernel here")


# speedup vs baseline: 1.4093x; 1.4093x over previous
import functools
import math

import jax
import jax.numpy as jnp
from jax import lax
from jax.experimental import pallas as pl
from jax.experimental.pallas import tpu as pltpu
from jax.experimental.pallas import tpu_sc as plsc

L = 16
IDXW = 128
R = 256
KPC = R // IDXW
NBUF = 4
LOOK = 3
NC, NS = 2, 16


@functools.lru_cache(maxsize=None)
def _build(b_flat: int, seq: int, dim: int):
    nw = NC * NS
    b_per_w = b_flat // nw
    nchunks = b_per_w // R
    assert b_per_w * nw == b_flat and nchunks * R == b_per_w
    assert b_per_w % seq == 0
    assert dim == 2 * L
    pr_rows = seq - math.gcd(R, seq) + R

    mesh = plsc.VectorSubcoreMesh(core_axis_name="c", subcore_axis_name="s",
                                  num_cores=NC, num_subcores=NS)
    scratch = (
        [pltpu.VMEM((KPC, IDXW), jnp.int32) for _ in range(NBUF)]
        + [pltpu.VMEM((R, dim), jnp.float32) for _ in range(NBUF)]
        + [pltpu.VMEM((pr_rows, dim), jnp.float32)]
        + [pltpu.SemaphoreType.DMA for _ in range(2 * NBUF)]
    )

    @functools.partial(
        pl.kernel,
        out_type=jax.ShapeDtypeStruct((b_flat, dim), jnp.float32),
        mesh=mesh,
        scratch_types=scratch,
        compiler_params=pltpu.CompilerParams(use_tc_tiling_on_sc=False),
    )
    def kern(idx_hbm, tok_hbm, pos_hbm, out_hbm, *sc):
        idxv = sc[0:NBUF]
        buf = sc[NBUF:2 * NBUF]
        pos_rep = sc[2 * NBUF]
        gsem = sc[2 * NBUF + 1:3 * NBUF + 1]
        osem = sc[3 * NBUF + 1:4 * NBUF + 1]

        wid = lax.axis_index("s") * NC + lax.axis_index("c")
        row0 = wid * b_per_w
        irow0 = wid * (b_per_w // IDXW)

        for t in range(pr_rows // seq):
            pltpu.sync_copy(pos_hbm, pos_rep.at[pl.ds(t * seq, seq)])
        if pr_rows % seq:
            pltpu.sync_copy(pos_hbm.at[pl.ds(0, pr_rows % seq)],
                            pos_rep.at[pl.ds((pr_rows // seq) * seq,
                                             pr_rows % seq)])

        def issue(g, b):
            pltpu.sync_copy(idx_hbm.at[pl.ds(irow0 + g * KPC, KPC)], idxv[b])
            for j in range(KPC):
                pltpu.async_copy(tok_hbm.at[idxv[b].at[j]],
                                 buf[b].at[pl.ds(j * IDXW, IDXW)], gsem[b])

        def gather_wait(b):
            for j in range(KPC):
                pltpu.make_async_copy(tok_hbm.at[idxv[b].at[j]],
                                      buf[b].at[pl.ds(j * IDXW, IDXW)],
                                      gsem[b]).wait()

        def out_wait(g, b):
            pltpu.make_async_copy(buf[b], out_hbm.at[pl.ds(row0 + g * R, R)],
                                  osem[b]).wait()

        for b in range(LOOK):
            issue(b, b)

        def chunk_group(i, _):
            for b in range(NBUF):
                g = i * NBUF + b
                gather_wait(b)
                ph = lax.rem(g * R, seq)

                def add_pos(r2, _, b=b, ph=ph):
                    r = r2 * 2
                    p00 = pos_rep[ph + r, pl.ds(0, L)]
                    p01 = pos_rep[ph + r, pl.ds(L, L)]
                    p10 = pos_rep[ph + r + 1, pl.ds(0, L)]
                    p11 = pos_rep[ph + r + 1, pl.ds(L, L)]
                    plsc.addupdate(buf[b].at[r, pl.ds(0, L)], p00)
                    plsc.addupdate(buf[b].at[r, pl.ds(L, L)], p01)
                    plsc.addupdate(buf[b].at[r + 1, pl.ds(0, L)], p10)
                    plsc.addupdate(buf[b].at[r + 1, pl.ds(L, L)], p11)
                    return 0

                lax.fori_loop(0, R // 2, add_pos, 0)
                pltpu.async_copy(buf[b], out_hbm.at[pl.ds(row0 + g * R, R)],
                                 osem[b])
                h = g + LOOK
                bh = (b + LOOK) % NBUF

                @pl.when(h < nchunks)
                def _(h=h, bh=bh):
                    @pl.when(h >= NBUF)
                    def _():
                        out_wait(h - NBUF, bh)
                    issue(h, bh)
            return 0

        lax.fori_loop(0, nchunks // NBUF, chunk_group, 0)
        for g in range(nchunks - NBUF, nchunks):
            out_wait(g, g % NBUF)

    return kern


def kernel(inputs, token_table, position_table):
    batch, seq = inputs.shape
    dim = token_table.shape[1]
    idx = inputs.reshape(-1).astype(jnp.int32).reshape(-1, IDXW)
    out = _build(batch * seq, seq, dim)(
        idx, token_table.astype(jnp.float32), position_table.astype(jnp.float32))
    return out.reshape(batch, seq, dim)
